# baseline (speedup 1.0000x reference)
import functools

import jax
import jax.numpy as jnp
from jax import lax
from jax.experimental import pallas as pl
from jax.experimental.pallas import tpu as pltpu
from jax.experimental.pallas import tpu_sc as plsc

_EMB_DIM = 128
_ROW = 2 * _EMB_DIM
_BATCH = 16384
_PI = 3.141592653589793
_EMB_RANGE = (6.0 + 2.0) / _EMB_DIM
_PHASE_SCALE = _PI / _EMB_RANGE

_info = plsc.get_sparse_core_info()
_NC = _info.num_cores
_NS = _info.num_subcores
_L = _info.num_lanes
_NW = _NC * _NS
_WH = _NW // 2

_TOTAL = 2 * _BATCH
_PER_W = _TOTAL // _NW
_C = 64
_NCHUNK = _PER_W // _C
_UNITS = _ROW // (2 * _L)


def _cs_body(r_ref, c_ref, s_ref):
    ph = r_ref[...] * _PHASE_SCALE
    c_ref[...] = jnp.cos(ph)
    s_ref[...] = jnp.sin(ph)


def _sqrt(m):
    m = m + 1e-35
    yi = plsc.bitcast(m, jnp.int32)
    yi = 0x5F3759DF - (yi >> 1)
    g = plsc.bitcast(yi, jnp.float32)
    hm = m * 0.5
    g = g * (1.5 - hm * g * g)
    g = g * (1.5 - hm * g * g)
    return m * g


def _take(x, idx):
    return jnp.take_along_axis(
        x, idx, axis=0, mode=lax.GatherScatterMode.PROMISE_IN_BOUNDS)


@functools.partial(
    pl.kernel,
    out_type=(
        jax.ShapeDtypeStruct((_BATCH,), jnp.float32),
        jax.ShapeDtypeStruct((_BATCH,), jnp.float32),
    ),
    mesh=plsc.VectorSubcoreMesh(core_axis_name="c", subcore_axis_name="s"),
    compiler_params=pltpu.CompilerParams(
        use_tc_tiling_on_sc=False, needs_layout_passes=False),
    scratch_types=[
        pltpu.VMEM((_C, _EMB_DIM), jnp.float32),
        pltpu.VMEM((_C, _EMB_DIM), jnp.float32),
        pltpu.VMEM((_C, _EMB_DIM), jnp.float32),
        pltpu.VMEM((_C, _EMB_DIM), jnp.float32),
        pltpu.VMEM((_C, _EMB_DIM), jnp.float32),
        pltpu.VMEM((_C, _EMB_DIM), jnp.float32),
        pltpu.VMEM((_C, _EMB_DIM), jnp.float32),
        pltpu.VMEM((_C, _EMB_DIM), jnp.float32),
        pltpu.VMEM((_C, _EMB_DIM), jnp.float32),
        pltpu.VMEM((_C, _EMB_DIM), jnp.float32),
        pltpu.VMEM((_C, _EMB_DIM), jnp.float32),
        pltpu.VMEM((_C, _EMB_DIM), jnp.float32),
        pltpu.VMEM((_PER_W,), jnp.int32),
        pltpu.VMEM((_PER_W,), jnp.int32),
        pltpu.VMEM((_PER_W,), jnp.int32),
        pltpu.VMEM((_PER_W,), jnp.int32),
        pltpu.VMEM((_PER_W,), jnp.int32),
        pltpu.VMEM((_L, _L + 1), jnp.float32),
        pltpu.VMEM((_PER_W,), jnp.float32),
        pltpu.SemaphoreType.DMA,
        pltpu.SemaphoreType.DMA,
    ],
)
def _sc_score(ent2, at, bt,
              ph1, ph2, pr, pt1, pt2, nh1, nh2, nr, nt1, nt2,
              out_p, out_n,
              ehl0, ehl1, ehh0, ehh1, etl0, etl1, eth0, eth1,
              cc0, cc1, ss0, ss1, h1idx, h2idx, t1idx, t2idx, ridx,
              stage, ob, sem0, sem1):
    wid = lax.axis_index("s") * _NC + lax.axis_index("c")
    is_pos = wid < _WH
    loff = pl.multiple_of(
        lax.select(is_pos, wid * _PER_W, (wid - _WH) * _PER_W), _PER_W)
    ehls = (ehl0, ehl1)
    ehhs = (ehh0, ehh1)
    etls = (etl0, etl1)
    eths = (eth0, eth1)
    ccs = (cc0, cc1)
    sss = (ss0, ss1)
    sems = (sem0, sem1)

    @pl.when(is_pos)
    def _():
        pltpu.sync_copy(ph1.at[pl.ds(loff, _PER_W)], h1idx)
        pltpu.sync_copy(ph2.at[pl.ds(loff, _PER_W)], h2idx)
        pltpu.sync_copy(pr.at[pl.ds(loff, _PER_W)], ridx)
        pltpu.sync_copy(pt1.at[pl.ds(loff, _PER_W)], t1idx)
        pltpu.sync_copy(pt2.at[pl.ds(loff, _PER_W)], t2idx)

    @pl.when(jnp.logical_not(is_pos))
    def _():
        pltpu.sync_copy(nh1.at[pl.ds(loff, _PER_W)], h1idx)
        pltpu.sync_copy(nh2.at[pl.ds(loff, _PER_W)], h2idx)
        pltpu.sync_copy(nr.at[pl.ds(loff, _PER_W)], ridx)
        pltpu.sync_copy(nt1.at[pl.ds(loff, _PER_W)], t1idx)
        pltpu.sync_copy(nt2.at[pl.ds(loff, _PER_W)], t2idx)

    def fire(g, b):
        o = pl.multiple_of(g * _C, _C)
        pltpu.async_copy(ent2.at[h1idx.at[pl.ds(o, _C)]], ehls[b], sems[b])
        pltpu.async_copy(ent2.at[h2idx.at[pl.ds(o, _C)]], ehhs[b], sems[b])
        pltpu.async_copy(ent2.at[t1idx.at[pl.ds(o, _C)]], etls[b], sems[b])
        pltpu.async_copy(ent2.at[t2idx.at[pl.ds(o, _C)]], eths[b], sems[b])
        pltpu.async_copy(at.at[ridx.at[pl.ds(o, _C)]], ccs[b], sems[b])
        pltpu.async_copy(bt.at[ridx.at[pl.ds(o, _C)]], sss[b], sems[b])

    def drain(b):
        pltpu.make_async_copy(ent2.at[pl.ds(0, _C)], ehls[b], sems[b]).wait()
        pltpu.make_async_copy(ent2.at[pl.ds(0, _C)], ehhs[b], sems[b]).wait()
        pltpu.make_async_copy(ent2.at[pl.ds(0, _C)], etls[b], sems[b]).wait()
        pltpu.make_async_copy(ent2.at[pl.ds(0, _C)], eths[b], sems[b]).wait()
        pltpu.make_async_copy(at.at[pl.ds(0, _C)], ccs[b], sems[b]).wait()
        pltpu.make_async_copy(at.at[pl.ds(0, _C)], sss[b], sems[b]).wait()

    iot = lax.iota(jnp.int32, _L)
    pswap = iot ^ 1
    pev = (iot % (_L // 2)) * 2
    pod = pev + 1
    mask_lo = iot < (_L // 2)
    dlo = iot >> 1
    dhi = dlo + (_L // 2)
    altv = jnp.where(iot % 2 == 0, -1.0, 1.0).astype(jnp.float32)
    rows = iot

    def compute(g, b):
        cc, ss = ccs[b], sss[b]
        ehalves = (ehls[b], ehhs[b])
        thalves = (etls[b], eths[b])

        for grp in range(_C // _L):
            def tbody(il, _):
                i = grp * _L + il
                acc = jnp.zeros((_L,), jnp.float32)
                for u in range(_UNITS):
                    eh = ehalves[u // (_UNITS // 2)]
                    et = thalves[u // (_UNITS // 2)]
                    eo0 = (2 * _L * u) % _EMB_DIM
                    eo1 = eo0 + _L
                    x0 = eh[i, pl.ds(eo0, _L)]
                    x1 = eh[i, pl.ds(eo1, _L)]
                    cu = cc[i, pl.ds(u * _L, _L)]
                    su = ss[i, pl.ds(u * _L, _L)]
                    a0 = _take(cu, dlo)
                    a1 = _take(cu, dhi)
                    b0 = _take(su, dlo) * altv
                    b1 = _take(su, dhi) * altv
                    t0 = et[i, pl.ds(eo0, _L)]
                    t1 = et[i, pl.ds(eo1, _L)]
                    d0 = x0 * a0 + _take(x0, pswap) * b0 - t0
                    d1 = x1 * a1 + _take(x1, pswap) * b1 - t1
                    q0 = d0 * d0
                    q1 = d1 * d1
                    me = jnp.where(mask_lo, _take(q0, pev), _take(q1, pev))
                    mo = jnp.where(mask_lo, _take(q0, pod), _take(q1, pod))
                    acc = acc + _sqrt(me + mo)
                stage[il, pl.ds(0, _L)] = acc
                return 0

            lax.fori_loop(0, _L, tbody, 0)
            score = plsc.load_gather(stage, [rows, iot * 0])
            for j in range(1, _L):
                score = score + plsc.load_gather(stage, [rows, iot * 0 + j])
            o = pl.multiple_of(g * _C + grp * _L, _L)
            ob[pl.ds(o, _L)] = score

    fire(0, 0)

    def pair_body(k, _):
        g0 = k * 2
        fire(g0 + 1, 1)
        drain(0)
        compute(g0, 0)
        fire(g0 + 2, 0)
        drain(1)
        compute(g0 + 1, 1)
        return 0

    lax.fori_loop(0, _NCHUNK // 2 - 1, pair_body, 0)
    fire(_NCHUNK - 1, 1)
    drain(0)
    compute(_NCHUNK - 2, 0)
    drain(1)
    compute(_NCHUNK - 1, 1)

    @pl.when(is_pos)
    def _():
        pltpu.sync_copy(ob, out_p.at[pl.ds(loff, _PER_W)])

    @pl.when(jnp.logical_not(is_pos))
    def _():
        pltpu.sync_copy(ob, out_n.at[pl.ds(loff, _PER_W)])


def _split_rows(idx):
    j1 = ((idx >> 3) << 4) | (idx & 7)
    return j1, j1 + 8


def kernel(entity_emb, relation_emb, pos_h, pos_r, pos_t, neg_h, neg_r, neg_t):
    nent = entity_emb.shape[0]
    nrel, dim = relation_emb.shape
    cs = pl.pallas_call(
        _cs_body,
        out_shape=(
            jax.ShapeDtypeStruct((nrel, dim), jnp.float32),
            jax.ShapeDtypeStruct((nrel, dim), jnp.float32),
        ),
    )
    a_t, b_t = cs(relation_emb)
    ent2 = (entity_emb.reshape(nent // 8, 8, 2, _EMB_DIM)
            .transpose(0, 2, 1, 3)
            .reshape(2 * nent, _EMB_DIM))
    i32 = jnp.int32
    ph1, ph2 = _split_rows(pos_h.astype(i32))
    pt1, pt2 = _split_rows(pos_t.astype(i32))
    nh1, nh2 = _split_rows(neg_h.astype(i32))
    nt1, nt2 = _split_rows(neg_t.astype(i32))
    return _sc_score(ent2, a_t, b_t,
                     ph1, ph2, pos_r.astype(i32), pt1, pt2,
                     nh1, nh2, neg_r.astype(i32), nt1, nt2)


# speedup vs baseline: 1.1650x; 1.1650x over previous
import functools

import jax
import jax.numpy as jnp
from jax import lax
from jax.experimental import pallas as pl
from jax.experimental.pallas import tpu as pltpu
from jax.experimental.pallas import tpu_sc as plsc

_EMB_DIM = 128
_ROW = 2 * _EMB_DIM
_BATCH = 16384
_PI = 3.141592653589793
_EMB_RANGE = (6.0 + 2.0) / _EMB_DIM
_PHASE_SCALE = _PI / _EMB_RANGE

_info = plsc.get_sparse_core_info()
_NC = _info.num_cores
_NS = _info.num_subcores
_L = _info.num_lanes
_NW = _NC * _NS
_WH = _NW // 2

_TOTAL = 2 * _BATCH
_PER_W = _TOTAL // _NW
_C = 32
_NCHUNK = _PER_W // _C
_UNITS = _ROW // (2 * _L)


def _ab_body(r2_ref, a_ref, b_ref):
    ph = r2_ref[...] * _PHASE_SCALE
    a_ref[...] = jnp.cos(ph)
    col = lax.broadcasted_iota(jnp.int32, ph.shape, 1)
    alt = jnp.where(col % 2 == 0, -1.0, 1.0).astype(jnp.float32)
    b_ref[...] = jnp.sin(ph) * alt


def _sqrt(m):
    m = m + 1e-35
    yi = plsc.bitcast(m, jnp.int32)
    yi = 0x5F3759DF - (yi >> 1)
    g = plsc.bitcast(yi, jnp.float32)
    hm = m * 0.5
    g = g * (1.5 - hm * g * g)
    return m * g


def _take(x, idx):
    return jnp.take_along_axis(
        x, idx, axis=0, mode=lax.GatherScatterMode.PROMISE_IN_BOUNDS)


@functools.partial(
    pl.kernel,
    out_type=(
        jax.ShapeDtypeStruct((_BATCH,), jnp.float32),
        jax.ShapeDtypeStruct((_BATCH,), jnp.float32),
    ),
    mesh=plsc.VectorSubcoreMesh(core_axis_name="c", subcore_axis_name="s"),
    compiler_params=pltpu.CompilerParams(
        use_tc_tiling_on_sc=False, needs_layout_passes=False),
    scratch_types=[
        pltpu.VMEM((_C, _EMB_DIM), jnp.float32),
        pltpu.VMEM((_C, _EMB_DIM), jnp.float32),
        pltpu.VMEM((_C, _EMB_DIM), jnp.float32),
        pltpu.VMEM((_C, _EMB_DIM), jnp.float32),
        pltpu.VMEM((_C, _EMB_DIM), jnp.float32),
        pltpu.VMEM((_C, _EMB_DIM), jnp.float32),
        pltpu.VMEM((_C, _EMB_DIM), jnp.float32),
        pltpu.VMEM((_C, _EMB_DIM), jnp.float32),
        pltpu.VMEM((_C, _ROW), jnp.float32),
        pltpu.VMEM((_C, _ROW), jnp.float32),
        pltpu.VMEM((_C, _ROW), jnp.float32),
        pltpu.VMEM((_C, _ROW), jnp.float32),
        pltpu.VMEM((_PER_W,), jnp.int32),
        pltpu.VMEM((_PER_W,), jnp.int32),
        pltpu.VMEM((_PER_W,), jnp.int32),
        pltpu.VMEM((_PER_W,), jnp.int32),
        pltpu.VMEM((_PER_W,), jnp.int32),
        pltpu.VMEM((_L, _L + 1), jnp.float32),
        pltpu.VMEM((_PER_W,), jnp.float32),
        pltpu.SemaphoreType.DMA,
        pltpu.SemaphoreType.DMA,
    ],
)
def _sc_score(ent2, at, bt,
              ph1, ph2, pr, pt1, pt2, nh1, nh2, nr, nt1, nt2,
              out_p, out_n,
              ehl0, ehl1, ehh0, ehh1, etl0, etl1, eth0, eth1,
              aa0, aa1, bb0, bb1, h1idx, h2idx, t1idx, t2idx, ridx,
              stage, ob, sem0, sem1):
    wid = lax.axis_index("s") * _NC + lax.axis_index("c")
    is_pos = wid < _WH
    loff = pl.multiple_of(
        lax.select(is_pos, wid * _PER_W, (wid - _WH) * _PER_W), _PER_W)
    ehls = (ehl0, ehl1)
    ehhs = (ehh0, ehh1)
    etls = (etl0, etl1)
    eths = (eth0, eth1)
    aas = (aa0, aa1)
    bbs = (bb0, bb1)
    sems = (sem0, sem1)

    @pl.when(is_pos)
    def _():
        pltpu.sync_copy(ph1.at[pl.ds(loff, _PER_W)], h1idx)
        pltpu.sync_copy(ph2.at[pl.ds(loff, _PER_W)], h2idx)
        pltpu.sync_copy(pr.at[pl.ds(loff, _PER_W)], ridx)
        pltpu.sync_copy(pt1.at[pl.ds(loff, _PER_W)], t1idx)
        pltpu.sync_copy(pt2.at[pl.ds(loff, _PER_W)], t2idx)

    @pl.when(jnp.logical_not(is_pos))
    def _():
        pltpu.sync_copy(nh1.at[pl.ds(loff, _PER_W)], h1idx)
        pltpu.sync_copy(nh2.at[pl.ds(loff, _PER_W)], h2idx)
        pltpu.sync_copy(nr.at[pl.ds(loff, _PER_W)], ridx)
        pltpu.sync_copy(nt1.at[pl.ds(loff, _PER_W)], t1idx)
        pltpu.sync_copy(nt2.at[pl.ds(loff, _PER_W)], t2idx)

    def fire(g, b):
        o = pl.multiple_of(g * _C, _C)
        pltpu.async_copy(ent2.at[h1idx.at[pl.ds(o, _C)]], ehls[b], sems[b])
        pltpu.async_copy(ent2.at[h2idx.at[pl.ds(o, _C)]], ehhs[b], sems[b])
        pltpu.async_copy(ent2.at[t1idx.at[pl.ds(o, _C)]], etls[b], sems[b])
        pltpu.async_copy(ent2.at[t2idx.at[pl.ds(o, _C)]], eths[b], sems[b])
        pltpu.async_copy(at.at[ridx.at[pl.ds(o, _C)]], aas[b], sems[b])
        pltpu.async_copy(bt.at[ridx.at[pl.ds(o, _C)]], bbs[b], sems[b])

    def drain(b):
        pltpu.make_async_copy(ent2.at[pl.ds(0, _C)], ehls[b], sems[b]).wait()
        pltpu.make_async_copy(ent2.at[pl.ds(0, _C)], ehhs[b], sems[b]).wait()
        pltpu.make_async_copy(ent2.at[pl.ds(0, _C)], etls[b], sems[b]).wait()
        pltpu.make_async_copy(ent2.at[pl.ds(0, _C)], eths[b], sems[b]).wait()
        pltpu.make_async_copy(at.at[pl.ds(0, _C)], aas[b], sems[b]).wait()
        pltpu.make_async_copy(at.at[pl.ds(0, _C)], bbs[b], sems[b]).wait()

    iot = lax.iota(jnp.int32, _L)
    pswap = iot ^ 1
    pev = (iot % (_L // 2)) * 2
    pod = pev + 1
    mask_lo = iot < (_L // 2)
    rows = iot

    def compute(g, b):
        aa, bb = aas[b], bbs[b]
        ehalves = (ehls[b], ehhs[b])
        thalves = (etls[b], eths[b])

        for grp in range(_C // _L):
            def tbody(il, _):
                i = grp * _L + il
                acc = jnp.zeros((_L,), jnp.float32)
                for u in range(_UNITS):
                    eh = ehalves[u // (_UNITS // 2)]
                    et = thalves[u // (_UNITS // 2)]
                    eo0 = (2 * _L * u) % _EMB_DIM
                    eo1 = eo0 + _L
                    o0 = 2 * _L * u
                    o1 = o0 + _L
                    x0 = eh[i, pl.ds(eo0, _L)]
                    x1 = eh[i, pl.ds(eo1, _L)]
                    a0 = aa[i, pl.ds(o0, _L)]
                    a1 = aa[i, pl.ds(o1, _L)]
                    b0 = bb[i, pl.ds(o0, _L)]
                    b1 = bb[i, pl.ds(o1, _L)]
                    t0 = et[i, pl.ds(eo0, _L)]
                    t1 = et[i, pl.ds(eo1, _L)]
                    d0 = x0 * a0 + _take(x0, pswap) * b0 - t0
                    d1 = x1 * a1 + _take(x1, pswap) * b1 - t1
                    q0 = d0 * d0
                    q1 = d1 * d1
                    me = jnp.where(mask_lo, _take(q0, pev), _take(q1, pev))
                    mo = jnp.where(mask_lo, _take(q0, pod), _take(q1, pod))
                    acc = acc + _sqrt(me + mo)
                stage[il, pl.ds(0, _L)] = acc
                return 0

            lax.fori_loop(0, _L, tbody, 0)
            score = plsc.load_gather(stage, [rows, iot * 0])
            for j in range(1, _L):
                score = score + plsc.load_gather(stage, [rows, iot * 0 + j])
            o = pl.multiple_of(g * _C + grp * _L, _L)
            ob[pl.ds(o, _L)] = score

    fire(0, 0)

    def pair_body(k, _):
        g0 = k * 2
        fire(g0 + 1, 1)
        drain(0)
        compute(g0, 0)
        fire(g0 + 2, 0)
        drain(1)
        compute(g0 + 1, 1)
        return 0

    lax.fori_loop(0, _NCHUNK // 2 - 1, pair_body, 0)
    fire(_NCHUNK - 1, 1)
    drain(0)
    compute(_NCHUNK - 2, 0)
    drain(1)
    compute(_NCHUNK - 1, 1)

    @pl.when(is_pos)
    def _():
        pltpu.sync_copy(ob, out_p.at[pl.ds(loff, _PER_W)])

    @pl.when(jnp.logical_not(is_pos))
    def _():
        pltpu.sync_copy(ob, out_n.at[pl.ds(loff, _PER_W)])


def _split_rows(idx):
    j1 = ((idx >> 3) << 4) | (idx & 7)
    return j1, j1 + 8


def kernel(entity_emb, relation_emb, pos_h, pos_r, pos_t, neg_h, neg_r, neg_t):
    nent = entity_emb.shape[0]
    nrel, dim = relation_emb.shape
    rel2 = jnp.repeat(relation_emb, 2, axis=1)
    ab = pl.pallas_call(
        _ab_body,
        out_shape=(
            jax.ShapeDtypeStruct((nrel, 2 * dim), jnp.float32),
            jax.ShapeDtypeStruct((nrel, 2 * dim), jnp.float32),
        ),
    )
    a_t, b_t = ab(rel2)
    ent2 = (entity_emb.reshape(nent // 8, 8, 2, _EMB_DIM)
            .transpose(0, 2, 1, 3)
            .reshape(2 * nent, _EMB_DIM))
    i32 = jnp.int32
    ph1, ph2 = _split_rows(pos_h.astype(i32))
    pt1, pt2 = _split_rows(pos_t.astype(i32))
    nh1, nh2 = _split_rows(neg_h.astype(i32))
    nt1, nt2 = _split_rows(neg_t.astype(i32))
    return _sc_score(ent2, a_t, b_t,
                     ph1, ph2, pos_r.astype(i32), pt1, pt2,
                     nh1, nh2, neg_r.astype(i32), nt1, nt2)
